# baseline (speedup 1.0000x reference)
import functools

import jax
import jax.numpy as jnp
from jax import lax
from jax.experimental import pallas as pl
from jax.experimental.pallas import tpu as pltpu
from jax.experimental.pallas import tpu_sc as plsc

_MASK_RATIO = 0.15
_ROWS_PER_BLOCK = 496
_LANES = 16


def _build_mask_row_sc(ones_row, idx_padded):
    (seq,) = ones_row.shape
    (n_pad,) = idx_padded.shape
    mesh = plsc.VectorSubcoreMesh(core_axis_name="c", subcore_axis_name="s")

    @functools.partial(
        pl.kernel,
        out_type=jax.ShapeDtypeStruct((seq,), jnp.float32),
        mesh=mesh,
        scratch_types=[
            pltpu.VMEM((n_pad,), jnp.int32),
            pltpu.VMEM((seq,), jnp.float32),
        ],
        compiler_params=pltpu.CompilerParams(needs_layout_passes=False),
    )
    def sc_mask(ones_hbm, idx_hbm, out_hbm, idx_v, row_v):
        wid = lax.axis_index("s") * 2 + lax.axis_index("c")

        @pl.when(wid == 0)
        def _():
            pltpu.sync_copy(ones_hbm, row_v)
            pltpu.sync_copy(idx_hbm, idx_v)
            zeros = jnp.zeros((_LANES,), jnp.float32)

            def body(i, carry):
                idx = idx_v[pl.ds(i * _LANES, _LANES)]
                plsc.store_scatter(row_v, [idx], zeros)
                return carry

            lax.fori_loop(0, n_pad // _LANES, body, 0)
            pltpu.sync_copy(row_v, out_hbm)

    return sc_mask(ones_row, idx_padded)


def _mask_body(widx_ref, m0_ref, m1_ref, x_ref, o_ref):
    row = jnp.where(widx_ref[0] == 0, m0_ref[...], m1_ref[...])
    o_ref[...] = x_ref[...] * row


def kernel(x, window_idx):
    batch, chans, seq = x.shape
    n_mask = int(seq * _MASK_RATIO)

    perm = jax.random.permutation(jax.random.key(42), seq)
    mask_idx = perm[:n_mask].astype(jnp.int32)
    n_pad = ((n_mask + _LANES - 1) // _LANES) * _LANES
    idx_padded = jnp.concatenate(
        [mask_idx, jnp.broadcast_to(mask_idx[:1], (n_pad - n_mask,))]
    )

    mask0 = _build_mask_row_sc(jnp.ones((seq,), jnp.float32), idx_padded)
    mask0 = mask0.reshape(1, seq)
    mask1 = jnp.ones((seq,), jnp.float32).at[seq - 1].set(0.0).reshape(1, seq)

    rows = batch * chans
    x2 = x.reshape(rows, seq)
    widx = jnp.asarray(window_idx, jnp.int32).reshape(1)

    blk = _ROWS_PER_BLOCK

    out = pl.pallas_call(
        _mask_body,
        grid=(pl.cdiv(rows, blk),),
        in_specs=[
            pl.BlockSpec(memory_space=pltpu.SMEM),
            pl.BlockSpec((1, seq), lambda i: (0, 0)),
            pl.BlockSpec((1, seq), lambda i: (0, 0)),
            pl.BlockSpec((blk, seq), lambda i: (i, 0)),
        ],
        out_specs=pl.BlockSpec((blk, seq), lambda i: (i, 0)),
        out_shape=jax.ShapeDtypeStruct((rows, seq), x.dtype),
        compiler_params=pltpu.CompilerParams(
            dimension_semantics=("arbitrary",),
            vmem_limit_bytes=67108864,
        ),
    )(widx, mask0, mask1, x2)
    return out.reshape(batch, chans, seq)


# speedup vs baseline: 1.0619x; 1.0619x over previous
import jax
import jax.numpy as jnp
from jax.experimental import pallas as pl
from jax.experimental.pallas import tpu as pltpu

_MASK_RATIO = 0.15
_BLK = 320
_SLOTS = 6
_LOOKAHEAD = 2


def _mask_body(widx_ref, m0_ref, m1_ref, x_ref, o_ref, buf, insems, outsems):
    rows, seq = x_ref.shape
    row = jnp.where(widx_ref[0] == 0, m0_ref[...], m1_ref[...])

    steps = []
    r0 = 0
    while r0 < rows:
        steps.append((r0, min(_BLK, rows - r0)))
        r0 += _BLK
    n = len(steps)

    def in_cp(i):
        r, nr = steps[i]
        s = i % _SLOTS
        return pltpu.make_async_copy(
            x_ref.at[pl.ds(r, nr)], buf.at[s, pl.ds(0, nr)], insems.at[s])

    def out_cp(i):
        r, nr = steps[i]
        s = i % _SLOTS
        return pltpu.make_async_copy(
            buf.at[s, pl.ds(0, nr)], o_ref.at[pl.ds(r, nr)], outsems.at[s])

    for k in range(min(_LOOKAHEAD, n)):
        in_cp(k).start()

    for i in range(n):
        if i - (_SLOTS - _LOOKAHEAD) >= 0:
            out_cp(i - (_SLOTS - _LOOKAHEAD)).wait()
        if i + _LOOKAHEAD < n:
            in_cp(i + _LOOKAHEAD).start()
        in_cp(i).wait()
        s = i % _SLOTS
        nr = steps[i][1]
        buf[s, : nr] = buf[s, : nr] * row
        out_cp(i).start()

    for i in range(max(0, n - (_SLOTS - _LOOKAHEAD)), n):
        out_cp(i).wait()


def kernel(x, window_idx):
    batch, chans, seq = x.shape
    n_mask = int(seq * _MASK_RATIO)

    perm = jax.random.permutation(jax.random.key(42), seq)
    mask_idx = perm[:n_mask]
    mask0 = jnp.ones((seq,), jnp.float32).at[mask_idx].set(0.0).reshape(1, seq)
    mask1 = jnp.ones((seq,), jnp.float32).at[seq - 1].set(0.0).reshape(1, seq)

    rows = batch * chans
    x2 = x.reshape(rows, seq)
    widx = jnp.asarray(window_idx, jnp.int32).reshape(1)

    out = pl.pallas_call(
        _mask_body,
        in_specs=[
            pl.BlockSpec(memory_space=pltpu.SMEM),
            pl.BlockSpec(memory_space=pltpu.MemorySpace.VMEM),
            pl.BlockSpec(memory_space=pltpu.MemorySpace.VMEM),
            pl.BlockSpec(memory_space=pltpu.MemorySpace.HBM),
        ],
        out_specs=pl.BlockSpec(memory_space=pltpu.MemorySpace.HBM),
        out_shape=jax.ShapeDtypeStruct((rows, seq), x.dtype),
        scratch_shapes=[
            pltpu.VMEM((_SLOTS, _BLK, seq), jnp.float32),
            pltpu.SemaphoreType.DMA((_SLOTS,)),
            pltpu.SemaphoreType.DMA((_SLOTS,)),
        ],
        compiler_params=pltpu.CompilerParams(
            vmem_limit_bytes=67108864,
        ),
    )(widx, mask0, mask1, x2)
    return out.reshape(batch, chans, seq)
